# Initial kernel scaffold; baseline (speedup 1.0000x reference)
import jax
import jax.numpy as jnp
from jax.experimental import pallas as pl


def kernel(h0, h1, h2, nei1, nei2, att_intra1, att_intra2, fc_w, fc_b, att_inter):
    raise NotImplementedError("write your pallas kernel here")


# speedup vs baseline: 4.1210x; 4.1210x over previous
import functools

import jax
import jax.numpy as jnp
from jax import lax
from jax.experimental import pallas as pl
from jax.experimental.pallas import tpu as pltpu
from jax.experimental.pallas import tpu_sc as plsc

N, D, M, S = 10000, 128, 50000, 16

NC, NS = 2, 16
ROWS = N * S
RPS = ROWS // NS
CHUNK = 400
NCHUNK = RPS // CHUNK


def _sc_gather(h1, h2, nei1f, nei2f):
    mesh = plsc.VectorSubcoreMesh(core_axis_name="c", subcore_axis_name="s")

    @functools.partial(
        pl.kernel,
        mesh=mesh,
        out_type=(
            jax.ShapeDtypeStruct((ROWS, D), jnp.float32),
            jax.ShapeDtypeStruct((ROWS, D), jnp.float32),
        ),
        scratch_types=[
            pltpu.VMEM((CHUNK,), jnp.int32),
            pltpu.VMEM((CHUNK, D), jnp.float32),
            pltpu.SemaphoreType.DMA,
        ],
    )
    def body(h1_hbm, h2_hbm, n1_hbm, n2_hbm, g1_hbm, g2_hbm, idx_v, rows_v, sem):
        c = lax.axis_index("c")
        s = lax.axis_index("s")
        base0 = s * RPS

        def run(table, idxs, out):
            def step(k, carry):
                off = pl.multiple_of(base0 + k * CHUNK, 8)
                pltpu.sync_copy(idxs.at[pl.ds(off, CHUNK)], idx_v)
                pltpu.async_copy(table.at[idx_v], rows_v, sem).wait()
                pltpu.sync_copy(rows_v, out.at[pl.ds(off, CHUNK)])
                return carry

            lax.fori_loop(0, NCHUNK, step, 0)

        @pl.when(c == 0)
        def _():
            run(h1_hbm, n1_hbm, g1_hbm)

        @pl.when(c == 1)
        def _():
            run(h2_hbm, n2_hbm, g2_hbm)

    return body(h1, h2, nei1f, nei2f)


BN = 400
GRID = N // BN


def _attn_body(h0_ref, g1_ref, g2_ref, a1r_ref, a1n_ref, a2r_ref, a2n_ref,
               fcw_ref, fcb_ref, e1_ref, e2_ref, sp_ref):
    @pl.when(pl.program_id(0) == 0)
    def _():
        sp_ref[...] = jnp.zeros_like(sp_ref)

    h0b = h0_ref[...]
    fcw = fcw_ref[...]
    fcb = fcb_ref[...]

    for v, (g_ref, ar_ref, an_ref, e_ref) in enumerate(
            ((g1_ref, a1r_ref, a1n_ref, e1_ref),
             (g2_ref, a2r_ref, a2n_ref, e2_ref))):
        g3 = g_ref[...].reshape(BN, S, D)
        c = jnp.sum(h0b * ar_ref[...], axis=1, keepdims=True)
        d = jnp.sum(g3 * an_ref[...].reshape(1, 1, D), axis=2)
        logits = c + d
        logits = jnp.where(logits > 0, logits, 0.01 * logits)
        m = jnp.max(logits, axis=1, keepdims=True)
        ex = jnp.exp(logits - m)
        w = ex / jnp.sum(ex, axis=1, keepdims=True)
        e = jnp.sum(w[:, :, None] * g3, axis=1)
        e = jnp.where(e > 0, e, jnp.exp(jnp.minimum(e, 0.0)) - 1.0)
        e_ref[...] = e
        t = jnp.tanh(
            jax.lax.dot_general(e, fcw, (((1,), (1,)), ((), ())),
                                preferred_element_type=jnp.float32) + fcb)
        sp_ref[v:v + 1, :] += jnp.sum(t, axis=0, keepdims=True)


def _attention(h0, g1, g2, a1r, a1n, a2r, a2n, fc_w, fc_b):
    return pl.pallas_call(
        _attn_body,
        grid=(GRID,),
        in_specs=[
            pl.BlockSpec((BN, D), lambda i: (i, 0)),
            pl.BlockSpec((BN * S, D), lambda i: (i, 0)),
            pl.BlockSpec((BN * S, D), lambda i: (i, 0)),
            pl.BlockSpec((1, D), lambda i: (0, 0)),
            pl.BlockSpec((1, D), lambda i: (0, 0)),
            pl.BlockSpec((1, D), lambda i: (0, 0)),
            pl.BlockSpec((1, D), lambda i: (0, 0)),
            pl.BlockSpec((D, D), lambda i: (0, 0)),
            pl.BlockSpec((1, D), lambda i: (0, 0)),
        ],
        out_specs=[
            pl.BlockSpec((BN, D), lambda i: (i, 0)),
            pl.BlockSpec((BN, D), lambda i: (i, 0)),
            pl.BlockSpec((2, D), lambda i: (0, 0)),
        ],
        out_shape=[
            jax.ShapeDtypeStruct((N, D), jnp.float32),
            jax.ShapeDtypeStruct((N, D), jnp.float32),
            jax.ShapeDtypeStruct((2, D), jnp.float32),
        ],
    )(h0, g1, g2, a1r, a1n, a2r, a2n, fc_w, fc_b)


def _combine_body(e1_ref, e2_ref, sp_ref, ai_ref, z_ref):
    b = jnp.sum(ai_ref[...] * sp_ref[...], axis=1, keepdims=True) / N
    m = jnp.max(b, axis=0, keepdims=True)
    ex = jnp.exp(b - m)
    beta = ex / jnp.sum(ex, axis=0, keepdims=True)
    z_ref[...] = (e1_ref[...] * beta[0:1, 0:1]
                  + e2_ref[...] * beta[1:2, 0:1])


def _combine(e1, e2, sp, att_inter):
    return pl.pallas_call(
        _combine_body,
        out_shape=jax.ShapeDtypeStruct((N, D), jnp.float32),
    )(e1, e2, sp, att_inter)


def kernel(h0, h1, h2, nei1, nei2, att_intra1, att_intra2, fc_w, fc_b, att_inter):
    nei1f = nei1.reshape(-1)
    nei2f = nei2.reshape(-1)
    g1, g2 = _sc_gather(h1, h2, nei1f, nei2f)
    a1r, a1n = att_intra1[:, :D], att_intra1[:, D:]
    a2r, a2n = att_intra2[:, :D], att_intra2[:, D:]
    e1, e2, sp = _attention(h0, g1, g2, a1r, a1n, a2r, a2n,
                            fc_w, fc_b.reshape(1, D))
    return _combine(e1, e2, sp, att_inter)
